# Initial kernel scaffold; baseline (speedup 1.0000x reference)
import jax
import jax.numpy as jnp
from jax.experimental import pallas as pl


def kernel(x, edge_index, eps, W1, b1, W2, b2, gamma, beta, pi, Wpi, bpi, Wout, bout):
    raise NotImplementedError("write your pallas kernel here")


# speedup vs baseline: 2.8615x; 2.8615x over previous
import functools

import jax
import jax.numpy as jnp
from jax import lax
from jax.experimental import pallas as pl
from jax.experimental.pallas import tpu as pltpu
from jax.experimental.pallas import tpu_sc as plsc

N = 10000
D = 128
H = 128
L = 4

_NC = 2
_NS = 16
_CH = 128
_NCHUNK = 158
_EPT = _CH * _NCHUNK
_EPAD = _EPT * _NS
_OWN = 5120
_ACC = 5248
_ZPT = _ACC // _NS
_CPT = _OWN // _NS


def _make_agg():
    mesh = plsc.VectorSubcoreMesh(core_axis_name="c", subcore_axis_name="s")

    @functools.partial(
        pl.kernel,
        out_type=jax.ShapeDtypeStruct((_NC, _OWN, H), jnp.float32),
        mesh=mesh,
        scratch_types=[
            pltpu.VMEM((_NCHUNK, _CH), jnp.int32),
            pltpu.VMEM((_NCHUNK, _CH), jnp.int32),
            pltpu.VMEM((_CH, H), jnp.float32),
            pltpu.VMEM((_CH, H), jnp.float32),
            pltpu.VMEM_SHARED((_ACC, H), jnp.float32),
            pltpu.SemaphoreType.DMA,
            pltpu.SemaphoreType.DMA,
        ],
    )
    def agg(h_hbm, src_hbm, dst_hbm, z_hbm, out_hbm,
            src_v, dst_v, buf0, buf1, acc, sem0, sem1):
        c = lax.axis_index("c")
        s = lax.axis_index("s")
        lo = c * _OWN
        pltpu.sync_copy(src_hbm.at[s], src_v)
        pltpu.sync_copy(dst_hbm.at[s], dst_v)
        pltpu.sync_copy(z_hbm.at[pl.ds(s * _ZPT, _ZPT)],
                        acc.at[pl.ds(s * _ZPT, _ZPT)])
        plsc.subcore_barrier()

        bufs = (buf0, buf1)
        sems = (sem0, sem1)
        pltpu.async_copy(h_hbm.at[src_v.at[0]], buf0, sem0)
        pltpu.async_copy(h_hbm.at[src_v.at[1]], buf1, sem1)

        def pair(i, carry):
            j0 = i * 2
            for k in range(2):
                j = j0 + k
                buf, sem = bufs[k], sems[k]
                pltpu.make_async_copy(h_hbm.at[src_v.at[j]], buf, sem).wait()
                for q in range(_CH // 16):
                    dv = dst_v[j, pl.ds(q * 16, 16)]
                    lv = dv - lo
                    ok = (lv >= 0) & (lv < _OWN)
                    dst_v[j, pl.ds(q * 16, 16)] = jnp.where(ok, lv, _OWN)
                pltpu.sync_copy(buf, acc.at[dst_v.at[j]], add=True)

                @pl.when(j + 2 < _NCHUNK)
                def _():
                    pltpu.async_copy(h_hbm.at[src_v.at[j + 2]], buf, sem)
            return carry

        lax.fori_loop(0, _NCHUNK // 2, pair, 0)
        plsc.subcore_barrier()
        pltpu.sync_copy(acc.at[pl.ds(s * _CPT, _CPT)],
                        out_hbm.at[c, pl.ds(s * _CPT, _CPT)])

    return agg


_agg = _make_agg()


def _tc_layer_body(sc_ref, h_ref, parts_ref, W1_ref, b1_ref, W2_ref, b2_ref,
                   g_ref, be_ref, hout_ref, rep_ref):
    h = h_ref[...]
    scale = sc_ref[0]
    agg = jnp.concatenate(
        [parts_ref[0, :, :], parts_ref[1, :N - _OWN, :]], axis=0)
    pooled = agg + scale * h
    h1 = jnp.maximum(
        jnp.dot(pooled, W1_ref[...], preferred_element_type=jnp.float32)
        + b1_ref[...], 0.0)
    h2 = (jnp.dot(h1, W2_ref[...], preferred_element_type=jnp.float32)
          + b2_ref[...])
    mu = jnp.mean(h2, axis=0, keepdims=True)
    var = jnp.mean((h2 - mu) ** 2, axis=0, keepdims=True)
    hbn = (h2 - mu) * lax.rsqrt(var + 1e-5) * g_ref[...] + be_ref[...]
    ho = jnp.maximum(hbn, 0.0)
    hout_ref[...] = ho
    rep_ref[...] = jnp.sum(ho, axis=0, keepdims=True)


_tc_layer = pl.pallas_call(
    _tc_layer_body,
    out_shape=[
        jax.ShapeDtypeStruct((N, H), jnp.float32),
        jax.ShapeDtypeStruct((1, H), jnp.float32),
    ],
    in_specs=[pl.BlockSpec(memory_space=pltpu.SMEM)]
    + [pl.BlockSpec(memory_space=pltpu.VMEM)] * 8,
)


def _head_body(x_ref, reps_ref, pi_ref, Wpi_ref, bpi_ref, Wout_ref, bout_ref,
               o_ref):
    rep0 = jnp.sum(x_ref[...], axis=0, keepdims=True)
    pi_emb = jnp.maximum(
        jnp.dot(pi_ref[...], Wpi_ref[...], preferred_element_type=jnp.float32)
        + bpi_ref[...], 0.0)
    acc = jnp.dot(rep0, Wout_ref[0:D, :], preferred_element_type=jnp.float32)
    for l in range(L):
        acc = acc + jnp.dot(
            reps_ref[l:l + 1, :],
            Wout_ref[D + H * l:D + H * (l + 1), :],
            preferred_element_type=jnp.float32)
    acc = acc + jnp.dot(pi_emb, Wout_ref[D + H * L:, :],
                        preferred_element_type=jnp.float32)
    o_ref[...] = acc + bout_ref[...]


_head = pl.pallas_call(
    _head_body,
    out_shape=jax.ShapeDtypeStruct((1, 2), jnp.float32),
)


def kernel(x, edge_index, eps, W1, b1, W2, b2, gamma, beta, pi, Wpi, bpi,
           Wout, bout):
    src = edge_index[0]
    dst = edge_index[1]
    e = src.shape[0]
    pad = _EPAD - e
    srcp = jnp.concatenate(
        [src, jnp.zeros((pad,), jnp.int32)]).reshape(_NS, _NCHUNK, _CH)
    dstp = jnp.concatenate(
        [dst, jnp.full((pad,), N, jnp.int32)]).reshape(_NS, _NCHUNK, _CH)
    zeros = jnp.zeros((_ACC, H), jnp.float32)

    h = x
    reps = []
    for l in range(L):
        parts = _agg(h, srcp, dstp, zeros)
        scale = (1.0 + eps[l]).reshape(1)
        h, rep = _tc_layer(scale, h, parts, W1[l], b1[l].reshape(1, H),
                           W2[l], b2[l].reshape(1, H), gamma[l].reshape(1, H),
                           beta[l].reshape(1, H))
        reps.append(rep)

    repstack = jnp.concatenate(reps, axis=0)
    return _head(x, repstack, pi, Wpi, bpi.reshape(1, 16), Wout,
                 bout.reshape(1, 2))
